# baseline (speedup 1.0000x reference)
import functools

import jax
import jax.numpy as jnp
from jax import lax
from jax.experimental import pallas as pl
from jax.experimental.pallas import tpu as pltpu
from jax.experimental.pallas import tpu_sc as plsc

EMBED_DIM = 64
FEAT_DIM = 64
BATCH = 16384
H1 = 32
H2 = 16

_SC_INFO = plsc.get_sparse_core_info()
_NC = _SC_INFO.num_cores
_NS = _SC_INFO.num_subcores
_NW = _NC * _NS
_B_PER_W = BATCH // _NW

_sc_mesh = plsc.VectorSubcoreMesh(core_axis_name="c", subcore_axis_name="s")


@functools.partial(
    pl.kernel,
    mesh=_sc_mesh,
    out_type=jax.ShapeDtypeStruct((BATCH, EMBED_DIM), jnp.float32),
    scratch_types=[
        pltpu.VMEM((_B_PER_W,), jnp.int32),
        pltpu.VMEM((_B_PER_W, EMBED_DIM), jnp.float32),
        pltpu.SemaphoreType.DMA,
    ],
    compiler_params=pltpu.CompilerParams(use_tc_tiling_on_sc=False),
)
def _sc_gather(table_hbm, idx_hbm, out_hbm, idx_v, rows_v, sem):
    wid = lax.axis_index("s") * _NC + lax.axis_index("c")
    base = wid * _B_PER_W
    pltpu.sync_copy(idx_hbm.at[pl.ds(base, _B_PER_W)], idx_v)
    pltpu.async_copy(table_hbm.at[idx_v], rows_v, sem).wait()
    pltpu.sync_copy(rows_v, out_hbm.at[pl.ds(base, _B_PER_W)])


_BLK = 2048


def _mlp_body(uft_ref, emb_ref, w1t_ref, b1_ref, w2t_ref, b2_ref,
              wctt_ref, wcbt_ref, bc_ref, out_ref):
    f32 = jnp.float32
    h = lax.dot_general(w1t_ref[...], uft_ref[...], (((1,), (0,)), ((), ())),
                        preferred_element_type=f32)
    h = jnp.maximum(h + b1_ref[...], 0.0)
    h = lax.dot_general(w2t_ref[...], h, (((1,), (0,)), ((), ())),
                        preferred_element_type=f32)
    h = jnp.maximum(h + b2_ref[...], 0.0)
    y = lax.dot_general(wctt_ref[...], emb_ref[...], (((1,), (1,)), ((), ())),
                        preferred_element_type=f32)
    y = y + lax.dot_general(wcbt_ref[...], h, (((1,), (0,)), ((), ())),
                            preferred_element_type=f32)
    out_ref[...] = jnp.maximum(y + bc_ref[...], 0.0)


def _mlp(uft, emb_p, W1T, b1c, W2T, b2c, WcTopTp, WcBotT, bcc):
    grid = (BATCH // _BLK,)
    return pl.pallas_call(
        _mlp_body,
        grid=grid,
        in_specs=[
            pl.BlockSpec((FEAT_DIM, _BLK), lambda i: (0, i)),
            pl.BlockSpec((_BLK, EMBED_DIM), lambda i: (i, 0)),
            pl.BlockSpec((H1, FEAT_DIM), lambda i: (0, 0)),
            pl.BlockSpec((H1, 1), lambda i: (0, 0)),
            pl.BlockSpec((H2, H1), lambda i: (0, 0)),
            pl.BlockSpec((H2, 1), lambda i: (0, 0)),
            pl.BlockSpec((EMBED_DIM, EMBED_DIM), lambda i: (0, 0)),
            pl.BlockSpec((EMBED_DIM, H2), lambda i: (0, 0)),
            pl.BlockSpec((EMBED_DIM, 1), lambda i: (0, 0)),
        ],
        out_specs=pl.BlockSpec((EMBED_DIM, _BLK), lambda i: (0, i)),
        out_shape=jax.ShapeDtypeStruct((EMBED_DIM, BATCH), jnp.float32),
    )(uft, emb_p, W1T, b1c, W2T, b2c, WcTopTp, WcBotT, bcc)


def kernel(user_id, user_features, table, W1, b1, W2, b2, Wc, bc):
    idx = user_id.astype(jnp.int32)
    emb_p = _sc_gather(table, idx)
    WcTopT = Wc[:EMBED_DIM].T
    WcBotT = Wc[EMBED_DIM:].T
    outT = _mlp(user_features.T, emb_p, W1.T, b1.reshape(H1, 1), W2.T,
                b2.reshape(H2, 1), WcTopT, WcBotT, bc.reshape(EMBED_DIM, 1))
    return outT.T


# speedup vs baseline: 1.2268x; 1.2268x over previous
import functools

import jax
import jax.numpy as jnp
from jax import lax
from jax.experimental import pallas as pl
from jax.experimental.pallas import tpu as pltpu
from jax.experimental.pallas import tpu_sc as plsc

EMBED_DIM = 64
FEAT_DIM = 64
BATCH = 16384
H1 = 32
H2 = 16

_SC_INFO = plsc.get_sparse_core_info()
_NC = _SC_INFO.num_cores
_NS = _SC_INFO.num_subcores
_NW = _NC * _NS
_B_PER_W = BATCH // _NW
_HALF = _B_PER_W // 2
_BLK = 2048
_PACK_ROWS = BATCH // 2
_ROWS_PER_BLK = _BLK // 2
_W_PER_BLK = _BLK // _B_PER_W

_sc_mesh = plsc.VectorSubcoreMesh(core_axis_name="c", subcore_axis_name="s")


@functools.partial(
    pl.kernel,
    mesh=_sc_mesh,
    out_type=jax.ShapeDtypeStruct((_PACK_ROWS, 2 * EMBED_DIM), jnp.float32),
    scratch_types=[
        pltpu.VMEM((_HALF,), jnp.int32),
        pltpu.VMEM((_HALF,), jnp.int32),
        pltpu.VMEM((_HALF, EMBED_DIM), jnp.float32),
        pltpu.VMEM((_HALF, EMBED_DIM), jnp.float32),
        pltpu.SemaphoreType.DMA,
        pltpu.SemaphoreType.DMA,
    ],
    compiler_params=pltpu.CompilerParams(use_tc_tiling_on_sc=False),
)
def _sc_gather(table_hbm, idx_hbm, out_hbm, idx_a, idx_b, rows_a, rows_b,
               sem_a, sem_b):
    wid = lax.axis_index("s") * _NC + lax.axis_index("c")
    blk = wid // _W_PER_BLK
    q = wid % _W_PER_BLK
    base_a = blk * _BLK + q * _HALF
    base_b = base_a + _BLK // 2
    row0 = blk * _ROWS_PER_BLK + q * _HALF
    pltpu.sync_copy(idx_hbm.at[pl.ds(base_a, _HALF)], idx_a)
    pltpu.sync_copy(idx_hbm.at[pl.ds(base_b, _HALF)], idx_b)
    cp_a = pltpu.async_copy(table_hbm.at[idx_a], rows_a, sem_a)
    cp_b = pltpu.async_copy(table_hbm.at[idx_b], rows_b, sem_b)
    cp_a.wait()
    cp_b.wait()
    pltpu.sync_copy(rows_a, out_hbm.at[pl.ds(row0, _HALF), pl.ds(0, EMBED_DIM)])
    pltpu.sync_copy(rows_b,
                    out_hbm.at[pl.ds(row0, _HALF), pl.ds(EMBED_DIM, EMBED_DIM)])


def _mlp_body(uft_ref, emb_ref, w1t_ref, b1_ref, w2t_ref, b2_ref,
              wctt_ref, wcbt_ref, bc_ref, out_ref):
    f32 = jnp.float32
    h = lax.dot_general(w1t_ref[...], uft_ref[...], (((1,), (0,)), ((), ())),
                        preferred_element_type=f32)
    h = jnp.maximum(h + b1_ref[...], 0.0)
    h = lax.dot_general(w2t_ref[...], h, (((1,), (0,)), ((), ())),
                        preferred_element_type=f32)
    h = jnp.maximum(h + b2_ref[...], 0.0)
    emb = emb_ref[...]
    y_lo = lax.dot_general(wctt_ref[...], emb[:, :EMBED_DIM],
                           (((1,), (1,)), ((), ())),
                           preferred_element_type=f32)
    y_hi = lax.dot_general(wctt_ref[...], emb[:, EMBED_DIM:],
                           (((1,), (1,)), ((), ())),
                           preferred_element_type=f32)
    y = jnp.concatenate([y_lo, y_hi], axis=1)
    y = y + lax.dot_general(wcbt_ref[...], h, (((1,), (0,)), ((), ())),
                            preferred_element_type=f32)
    out_ref[...] = jnp.maximum(y + bc_ref[...], 0.0)


def _mlp(uft, emb_p, W1T, b1c, W2T, b2c, WcTopT, WcBotT, bcc):
    grid = (BATCH // _BLK,)
    return pl.pallas_call(
        _mlp_body,
        grid=grid,
        in_specs=[
            pl.BlockSpec((FEAT_DIM, _BLK), lambda i: (0, i)),
            pl.BlockSpec((_ROWS_PER_BLK, 2 * EMBED_DIM), lambda i: (i, 0)),
            pl.BlockSpec((H1, FEAT_DIM), lambda i: (0, 0)),
            pl.BlockSpec((H1, 1), lambda i: (0, 0)),
            pl.BlockSpec((H2, H1), lambda i: (0, 0)),
            pl.BlockSpec((H2, 1), lambda i: (0, 0)),
            pl.BlockSpec((EMBED_DIM, EMBED_DIM), lambda i: (0, 0)),
            pl.BlockSpec((EMBED_DIM, H2), lambda i: (0, 0)),
            pl.BlockSpec((EMBED_DIM, 1), lambda i: (0, 0)),
        ],
        out_specs=pl.BlockSpec((EMBED_DIM, _BLK), lambda i: (0, i)),
        out_shape=jax.ShapeDtypeStruct((EMBED_DIM, BATCH), jnp.float32),
    )(uft, emb_p, W1T, b1c, W2T, b2c, WcTopT, WcBotT, bcc)


def kernel(user_id, user_features, table, W1, b1, W2, b2, Wc, bc):
    idx = user_id.astype(jnp.int32)
    emb_p = _sc_gather(table, idx)
    WcTopT = Wc[:EMBED_DIM].T
    WcBotT = Wc[EMBED_DIM:].T
    outT = _mlp(user_features.T, emb_p, W1.T, b1.reshape(H1, 1), W2.T,
                b2.reshape(H2, 1), WcTopT, WcBotT, bc.reshape(EMBED_DIM, 1))
    return outT.T


# speedup vs baseline: 1.2951x; 1.0556x over previous
import functools

import jax
import jax.numpy as jnp
from jax import lax
from jax.experimental import pallas as pl
from jax.experimental.pallas import tpu as pltpu
from jax.experimental.pallas import tpu_sc as plsc

EMBED_DIM = 64
FEAT_DIM = 64
BATCH = 16384
H1 = 32
H2 = 16

_SC_INFO = plsc.get_sparse_core_info()
_NC = _SC_INFO.num_cores
_NS = _SC_INFO.num_subcores
_NW = _NC * _NS
_B_PER_W = BATCH // _NW
_HALF = _B_PER_W // 2
_BLK = 4096
_PACK_ROWS = BATCH // 2
_ROWS_PER_BLK = _BLK // 2
_W_PER_BLK = _BLK // _B_PER_W

_sc_mesh = plsc.VectorSubcoreMesh(core_axis_name="c", subcore_axis_name="s")


@functools.partial(
    pl.kernel,
    mesh=_sc_mesh,
    out_type=jax.ShapeDtypeStruct((_PACK_ROWS, 2 * EMBED_DIM), jnp.float32),
    scratch_types=[
        pltpu.VMEM((_HALF,), jnp.int32),
        pltpu.VMEM((_HALF,), jnp.int32),
        pltpu.VMEM((_HALF, EMBED_DIM), jnp.float32),
        pltpu.VMEM((_HALF, EMBED_DIM), jnp.float32),
        pltpu.SemaphoreType.DMA,
        pltpu.SemaphoreType.DMA,
    ],
    compiler_params=pltpu.CompilerParams(use_tc_tiling_on_sc=False),
)
def _sc_gather(table_hbm, idx_hbm, out_hbm, idx_a, idx_b, rows_a, rows_b,
               sem_a, sem_b):
    wid = lax.axis_index("s") * _NC + lax.axis_index("c")
    blk = wid // _W_PER_BLK
    q = wid % _W_PER_BLK
    base_a = blk * _BLK + q * _HALF
    base_b = base_a + _BLK // 2
    row0 = blk * _ROWS_PER_BLK + q * _HALF
    pltpu.sync_copy(idx_hbm.at[pl.ds(base_a, _HALF)], idx_a)
    pltpu.sync_copy(idx_hbm.at[pl.ds(base_b, _HALF)], idx_b)
    cp_a = pltpu.async_copy(table_hbm.at[idx_a], rows_a, sem_a)
    cp_b = pltpu.async_copy(table_hbm.at[idx_b], rows_b, sem_b)
    cp_a.wait()
    cp_b.wait()
    pltpu.sync_copy(rows_a, out_hbm.at[pl.ds(row0, _HALF), pl.ds(0, EMBED_DIM)])
    pltpu.sync_copy(rows_b,
                    out_hbm.at[pl.ds(row0, _HALF), pl.ds(EMBED_DIM, EMBED_DIM)])


def _mlp_body(uft_ref, emb_ref, w1t_ref, b1_ref, w2t_ref, b2_ref,
              wctt_ref, wcbt_ref, bc_ref, out_ref):
    f32 = jnp.float32
    h = lax.dot_general(w1t_ref[...], uft_ref[...], (((1,), (0,)), ((), ())),
                        preferred_element_type=f32)
    h = jnp.maximum(h + b1_ref[...], 0.0)
    h = lax.dot_general(w2t_ref[...], h, (((1,), (0,)), ((), ())),
                        preferred_element_type=f32)
    h = jnp.maximum(h + b2_ref[...], 0.0)
    emb = emb_ref[...]
    y_lo = lax.dot_general(wctt_ref[...], emb[:, :EMBED_DIM],
                           (((1,), (1,)), ((), ())),
                           preferred_element_type=f32)
    y_hi = lax.dot_general(wctt_ref[...], emb[:, EMBED_DIM:],
                           (((1,), (1,)), ((), ())),
                           preferred_element_type=f32)
    y = jnp.concatenate([y_lo, y_hi], axis=1)
    y = y + lax.dot_general(wcbt_ref[...], h, (((1,), (0,)), ((), ())),
                            preferred_element_type=f32)
    out_ref[...] = jnp.maximum(y + bc_ref[...], 0.0)


def _mlp(uft, emb_p, W1T, b1c, W2T, b2c, WcTopT, WcBotT, bcc):
    grid = (BATCH // _BLK,)
    return pl.pallas_call(
        _mlp_body,
        grid=grid,
        in_specs=[
            pl.BlockSpec((FEAT_DIM, _BLK), lambda i: (0, i)),
            pl.BlockSpec((_ROWS_PER_BLK, 2 * EMBED_DIM), lambda i: (i, 0)),
            pl.BlockSpec((H1, FEAT_DIM), lambda i: (0, 0)),
            pl.BlockSpec((H1, 1), lambda i: (0, 0)),
            pl.BlockSpec((H2, H1), lambda i: (0, 0)),
            pl.BlockSpec((H2, 1), lambda i: (0, 0)),
            pl.BlockSpec((EMBED_DIM, EMBED_DIM), lambda i: (0, 0)),
            pl.BlockSpec((EMBED_DIM, H2), lambda i: (0, 0)),
            pl.BlockSpec((EMBED_DIM, 1), lambda i: (0, 0)),
        ],
        out_specs=pl.BlockSpec((EMBED_DIM, _BLK), lambda i: (0, i)),
        out_shape=jax.ShapeDtypeStruct((EMBED_DIM, BATCH), jnp.float32),
    )(uft, emb_p, W1T, b1c, W2T, b2c, WcTopT, WcBotT, bcc)


def kernel(user_id, user_features, table, W1, b1, W2, b2, Wc, bc):
    idx = user_id.astype(jnp.int32)
    emb_p = _sc_gather(table, idx)
    WcTopT = Wc[:EMBED_DIM].T
    WcBotT = Wc[EMBED_DIM:].T
    outT = _mlp(user_features.T, emb_p, W1.T, b1.reshape(H1, 1), W2.T,
                b2.reshape(H2, 1), WcTopT, WcBotT, bc.reshape(EMBED_DIM, 1))
    return outT.T


# speedup vs baseline: 1.3307x; 1.0275x over previous
import functools

import jax
import jax.numpy as jnp
from jax import lax
from jax.experimental import pallas as pl
from jax.experimental.pallas import tpu as pltpu
from jax.experimental.pallas import tpu_sc as plsc

EMBED_DIM = 64
FEAT_DIM = 64
BATCH = 16384
H1 = 32
H2 = 16

_SC_INFO = plsc.get_sparse_core_info()
_NC = _SC_INFO.num_cores
_NS = _SC_INFO.num_subcores
_NW = _NC * _NS
_B_PER_W = BATCH // _NW
_HALF = _B_PER_W // 2
_BLK = 8192
_PACK_ROWS = BATCH // 2
_ROWS_PER_BLK = _BLK // 2
_W_PER_BLK = _BLK // _B_PER_W

_sc_mesh = plsc.VectorSubcoreMesh(core_axis_name="c", subcore_axis_name="s")


@functools.partial(
    pl.kernel,
    mesh=_sc_mesh,
    out_type=jax.ShapeDtypeStruct((_PACK_ROWS, 2 * EMBED_DIM), jnp.float32),
    scratch_types=[
        pltpu.VMEM((_HALF,), jnp.int32),
        pltpu.VMEM((_HALF,), jnp.int32),
        pltpu.VMEM((_HALF, EMBED_DIM), jnp.float32),
        pltpu.VMEM((_HALF, EMBED_DIM), jnp.float32),
        pltpu.SemaphoreType.DMA,
        pltpu.SemaphoreType.DMA,
    ],
    compiler_params=pltpu.CompilerParams(use_tc_tiling_on_sc=False),
)
def _sc_gather(table_hbm, idx_hbm, out_hbm, idx_a, idx_b, rows_a, rows_b,
               sem_a, sem_b):
    wid = lax.axis_index("s") * _NC + lax.axis_index("c")
    blk = wid // _W_PER_BLK
    q = wid % _W_PER_BLK
    base_a = blk * _BLK + q * _HALF
    base_b = base_a + _BLK // 2
    row0 = blk * _ROWS_PER_BLK + q * _HALF
    pltpu.sync_copy(idx_hbm.at[pl.ds(base_a, _HALF)], idx_a)
    pltpu.sync_copy(idx_hbm.at[pl.ds(base_b, _HALF)], idx_b)
    cp_a = pltpu.async_copy(table_hbm.at[idx_a], rows_a, sem_a)
    cp_b = pltpu.async_copy(table_hbm.at[idx_b], rows_b, sem_b)
    cp_a.wait()
    cp_b.wait()
    pltpu.sync_copy(rows_a, out_hbm.at[pl.ds(row0, _HALF), pl.ds(0, EMBED_DIM)])
    pltpu.sync_copy(rows_b,
                    out_hbm.at[pl.ds(row0, _HALF), pl.ds(EMBED_DIM, EMBED_DIM)])


def _mlp_body(uft_ref, emb_ref, w1t_ref, b1_ref, w2t_ref, b2_ref,
              wctt_ref, wcbt_ref, bc_ref, out_ref):
    f32 = jnp.float32
    h = lax.dot_general(w1t_ref[...], uft_ref[...], (((1,), (0,)), ((), ())),
                        preferred_element_type=f32)
    h = jnp.maximum(h + b1_ref[...], 0.0)
    h = lax.dot_general(w2t_ref[...], h, (((1,), (0,)), ((), ())),
                        preferred_element_type=f32)
    h = jnp.maximum(h + b2_ref[...], 0.0)
    emb = emb_ref[...]
    y_lo = lax.dot_general(wctt_ref[...], emb[:, :EMBED_DIM],
                           (((1,), (1,)), ((), ())),
                           preferred_element_type=f32)
    y_hi = lax.dot_general(wctt_ref[...], emb[:, EMBED_DIM:],
                           (((1,), (1,)), ((), ())),
                           preferred_element_type=f32)
    y = jnp.concatenate([y_lo, y_hi], axis=1)
    y = y + lax.dot_general(wcbt_ref[...], h, (((1,), (0,)), ((), ())),
                            preferred_element_type=f32)
    out_ref[...] = jnp.maximum(y + bc_ref[...], 0.0)


def _mlp(uft, emb_p, W1T, b1c, W2T, b2c, WcTopT, WcBotT, bcc):
    grid = (BATCH // _BLK,)
    return pl.pallas_call(
        _mlp_body,
        grid=grid,
        in_specs=[
            pl.BlockSpec((FEAT_DIM, _BLK), lambda i: (0, i)),
            pl.BlockSpec((_ROWS_PER_BLK, 2 * EMBED_DIM), lambda i: (i, 0)),
            pl.BlockSpec((H1, FEAT_DIM), lambda i: (0, 0)),
            pl.BlockSpec((H1, 1), lambda i: (0, 0)),
            pl.BlockSpec((H2, H1), lambda i: (0, 0)),
            pl.BlockSpec((H2, 1), lambda i: (0, 0)),
            pl.BlockSpec((EMBED_DIM, EMBED_DIM), lambda i: (0, 0)),
            pl.BlockSpec((EMBED_DIM, H2), lambda i: (0, 0)),
            pl.BlockSpec((EMBED_DIM, 1), lambda i: (0, 0)),
        ],
        out_specs=pl.BlockSpec((EMBED_DIM, _BLK), lambda i: (0, i)),
        out_shape=jax.ShapeDtypeStruct((EMBED_DIM, BATCH), jnp.float32),
    )(uft, emb_p, W1T, b1c, W2T, b2c, WcTopT, WcBotT, bcc)


def kernel(user_id, user_features, table, W1, b1, W2, b2, Wc, bc):
    idx = user_id.astype(jnp.int32)
    emb_p = _sc_gather(table, idx)
    WcTopT = Wc[:EMBED_DIM].T
    WcBotT = Wc[EMBED_DIM:].T
    outT = _mlp(user_features.T, emb_p, W1.T, b1.reshape(H1, 1), W2.T,
                b2.reshape(H2, 1), WcTopT, WcBotT, bc.reshape(EMBED_DIM, 1))
    return outT.T
